# baseline (speedup 1.0000x reference)
import functools

import jax
import jax.numpy as jnp
from jax import lax
from jax.experimental import pallas as pl
from jax.experimental.pallas import tpu as pltpu
from jax.experimental.pallas import tpu_sc as plsc

NC = 2
NS = 16
NW = NC * NS
L = 16

DP = 128
NBUF = 2

GROUP_STARTS = tuple(range(0, 192, 16)) + (184,)
RUNS = ((0, 128), (128, 72))


def kernel(input_ids, table):
    B, S = input_ids.shape
    V, D = table.shape
    rows_per_w = B // NW
    n_chunks = rows_per_w

    tpad = jnp.pad(table, ((0, 0), (0, DP - D)))

    mesh = plsc.VectorSubcoreMesh(core_axis_name="c", subcore_axis_name="s")

    @functools.partial(
        pl.kernel,
        mesh=mesh,
        out_type=jax.ShapeDtypeStruct((B, S, D), jnp.float32),
        scratch_types=[
            pltpu.VMEM((rows_per_w, S), jnp.int32),
            pltpu.VMEM((NBUF, S, DP), jnp.float32),
            pltpu.SemaphoreType.DMA,
            pltpu.SemaphoreType.DMA,
            pltpu.SemaphoreType.DMA,
            pltpu.SemaphoreType.DMA,
        ],
        compiler_params=pltpu.CompilerParams(
            needs_layout_passes=False, use_tc_tiling_on_sc=False
        ),
    )
    def emb_kernel(idx_hbm, table_hbm, out_hbm, idx_v, rows_v, sg0, sg1, so0, so1):
        wid = lax.axis_index("s") * NC + lax.axis_index("c")
        b0w = wid * rows_per_w
        sem_g = (sg0, sg1)
        sem_o = (so0, so1)

        pltpu.sync_copy(idx_hbm.at[pl.ds(b0w, rows_per_w)], idx_v)

        def start_gather(c, b):
            for off, n in RUNS:
                pltpu.async_copy(
                    table_hbm.at[idx_v.at[c, pl.ds(off, n)]],
                    rows_v.at[b].at[pl.ds(off, n)],
                    sem_g[b],
                )

        def wait_gather(b):
            for off, n in RUNS:
                pltpu.make_async_copy(
                    table_hbm.at[pl.ds(0, n)],
                    rows_v.at[b].at[pl.ds(off, n)],
                    sem_g[b],
                ).wait()

        def start_out(c, b):
            pltpu.async_copy(
                rows_v.at[b].at[:, pl.ds(0, D)],
                out_hbm.at[b0w + c],
                sem_o[b],
            )

        def wait_out(b):
            pltpu.make_async_copy(
                table_hbm.at[pl.ds(0, (S * D) // DP)],
                rows_v.at[b].at[:, pl.ds(0, D)],
                sem_o[b],
            ).wait()

        def fixup(c, b):
            m = None
            for off in GROUP_STARTS:
                iv = idx_v[c, pl.ds(off, L)]
                m = iv if m is None else jnp.minimum(m, iv)
            pad_cnt = plsc.all_reduce_population_count(m == 0)

            @pl.when(pad_cnt[0] != 0)
            def _fixup():
                zeros = jnp.zeros((L,), jnp.float32)
                lane = lax.iota(jnp.int32, L)

                def group_body(g, carry2):
                    off = jnp.minimum(g * L, S - L)
                    iv = idx_v[c, pl.ds(off, L)]
                    is_pad = iv == 0
                    gcnt = plsc.all_reduce_population_count(is_pad)

                    @pl.when(gcnt[0] != 0)
                    def _zero_rows():
                        srow = off + lane
                        for col in range(D):
                            plsc.store_scatter(
                                rows_v.at[b],
                                [srow, jnp.full((L,), col, jnp.int32)],
                                zeros,
                                mask=is_pad,
                            )

                    return carry2

                lax.fori_loop(0, len(GROUP_STARTS), group_body, 0)

        start_gather(0, 0)

        def pair_body(i0, carry):
            c0 = i0 * 2
            c1 = c0 + 1

            @pl.when(c0 > 0)
            def _():
                wait_out(1)
            start_gather(c1, 1)
            wait_gather(0)
            fixup(c0, 0)
            start_out(c0, 0)

            @pl.when(c1 < n_chunks - 1)
            def _():
                wait_out(0)
                start_gather(c1 + 1, 0)
            wait_gather(1)
            fixup(c1, 1)
            start_out(c1, 1)
            return carry

        lax.fori_loop(0, n_chunks // 2, pair_body, 0)
        wait_out(0)
        wait_out(1)

    return emb_kernel(input_ids, tpad)


# speedup vs baseline: 1.2365x; 1.2365x over previous
import functools

import jax
import jax.numpy as jnp
from jax import lax
from jax.experimental import pallas as pl
from jax.experimental.pallas import tpu as pltpu
from jax.experimental.pallas import tpu_sc as plsc

NC = 2
NS = 16
NW = NC * NS
L = 16

DP = 128
NBUF = 2

GROUP_STARTS = tuple(range(0, 192, 16)) + (184,)
RUNS = ((0, 128), (128, 72))


def kernel(input_ids, table):
    B, S = input_ids.shape
    V, D = table.shape
    rows_per_w = B // NW
    n_chunks = rows_per_w

    tpad = jnp.pad(table, ((0, 0), (0, DP - D)))

    mesh = plsc.VectorSubcoreMesh(core_axis_name="c", subcore_axis_name="s")

    @functools.partial(
        pl.kernel,
        mesh=mesh,
        out_type=jax.ShapeDtypeStruct((B, S, DP), jnp.float32),
        scratch_types=[
            pltpu.VMEM((rows_per_w, S), jnp.int32),
            pltpu.VMEM((NBUF, S, DP), jnp.float32),
            pltpu.SemaphoreType.DMA,
            pltpu.SemaphoreType.DMA,
            pltpu.SemaphoreType.DMA,
            pltpu.SemaphoreType.DMA,
        ],
        compiler_params=pltpu.CompilerParams(
            needs_layout_passes=False, use_tc_tiling_on_sc=False
        ),
    )
    def emb_kernel(idx_hbm, table_hbm, out_hbm, idx_v, rows_v, sg0, sg1, so0, so1):
        wid = lax.axis_index("s") * NC + lax.axis_index("c")
        b0w = wid * rows_per_w
        sem_g = (sg0, sg1)
        sem_o = (so0, so1)

        pltpu.sync_copy(idx_hbm.at[pl.ds(b0w, rows_per_w)], idx_v)

        def start_gather(c, b):
            for off, n in RUNS:
                pltpu.async_copy(
                    table_hbm.at[idx_v.at[c, pl.ds(off, n)]],
                    rows_v.at[b].at[pl.ds(off, n)],
                    sem_g[b],
                )

        def wait_gather(b):
            for off, n in RUNS:
                pltpu.make_async_copy(
                    table_hbm.at[pl.ds(0, n)],
                    rows_v.at[b].at[pl.ds(off, n)],
                    sem_g[b],
                ).wait()

        def start_out(c, b):
            pltpu.async_copy(
                rows_v.at[b],
                out_hbm.at[b0w + c],
                sem_o[b],
            )

        def wait_out(b):
            pltpu.make_async_copy(
                table_hbm.at[pl.ds(0, S)],
                rows_v.at[b],
                sem_o[b],
            ).wait()

        def fixup(c, b):
            m = None
            for off in GROUP_STARTS:
                iv = idx_v[c, pl.ds(off, L)]
                m = iv if m is None else jnp.minimum(m, iv)
            pad_cnt = plsc.all_reduce_population_count(m == 0)

            @pl.when(pad_cnt[0] != 0)
            def _fixup():
                zeros = jnp.zeros((L,), jnp.float32)
                lane = lax.iota(jnp.int32, L)

                def group_body(g, carry2):
                    off = jnp.minimum(g * L, S - L)
                    iv = idx_v[c, pl.ds(off, L)]
                    is_pad = iv == 0
                    gcnt = plsc.all_reduce_population_count(is_pad)

                    @pl.when(gcnt[0] != 0)
                    def _zero_rows():
                        srow = off + lane
                        for col in range(D):
                            plsc.store_scatter(
                                rows_v.at[b],
                                [srow, jnp.full((L,), col, jnp.int32)],
                                zeros,
                                mask=is_pad,
                            )

                    return carry2

                lax.fori_loop(0, len(GROUP_STARTS), group_body, 0)

        start_gather(0, 0)

        def pair_body(i0, carry):
            c0 = i0 * 2
            c1 = c0 + 1

            @pl.when(c0 > 0)
            def _():
                wait_out(1)
            start_gather(c1, 1)
            wait_gather(0)
            fixup(c0, 0)
            start_out(c0, 0)

            @pl.when(c1 < n_chunks - 1)
            def _():
                wait_out(0)
                start_gather(c1 + 1, 0)
            wait_gather(1)
            fixup(c1, 1)
            start_out(c1, 1)
            return carry

        lax.fori_loop(0, n_chunks // 2, pair_body, 0)
        wait_out(0)
        wait_out(1)

    return emb_kernel(input_ids, tpad)[:, :, :D]
